# Initial kernel scaffold; baseline (speedup 1.0000x reference)
import jax
import jax.numpy as jnp
from jax.experimental import pallas as pl


def kernel(sparse_idx, seq_idx, dense_vals, table):
    raise NotImplementedError("write your pallas kernel here")


# speedup vs baseline: 3.7327x; 3.7327x over previous
import functools

import jax
import jax.numpy as jnp
import numpy as np
from jax import lax
from jax.experimental import pallas as pl
from jax.experimental.pallas import tpu as pltpu
from jax.experimental.pallas import tpu_sc as plsc

B, V, D, NF, L = 4096, 100000, 64, 26, 50
NC, NS = 2, 16
NW = NC * NS
BPW = B // NW
SPC = BPW * NF // 128
NVR = D // 16


def _body(table, sp2d, seq2d, out_sp, out_pool,
          sidx_v, srows_v, seqidx_v, seqrows_v, pool_v, sem):
    c = lax.axis_index("c")
    s = lax.axis_index("s")
    w = s * NC + c

    def sp_chunk(i, carry):
        blk = w * SPC + i
        pltpu.sync_copy(sp2d.at[blk], sidx_v)
        pltpu.async_copy(table.at[sidx_v], srows_v, sem).wait()
        pltpu.sync_copy(srows_v, out_sp.at[pl.ds(blk * 128, 128)])
        return carry

    lax.fori_loop(0, SPC, sp_chunk, 0)

    pltpu.sync_copy(seq2d.at[pl.ds(w * BPW, BPW)], seqidx_v)
    scale = jnp.full((16,), np.float32(1.0 / L), jnp.float32)

    def row(j, carry):
        pltpu.async_copy(table.at[seqidx_v.at[j]], seqrows_v, sem).wait()

        def red(k, accs):
            return tuple(accs[d] + seqrows_v[k, pl.ds(d * 16, 16)]
                         for d in range(NVR))

        accs = lax.fori_loop(
            0, L, red, tuple(jnp.zeros((16,), jnp.float32)
                             for _ in range(NVR)))
        for d in range(NVR):
            pool_v[j, pl.ds(d * 16, 16)] = accs[d] * scale
        return carry

    lax.fori_loop(0, BPW, row, 0)
    pltpu.sync_copy(pool_v, out_pool.at[pl.ds(w * BPW, BPW)])


@jax.jit
def kernel(sparse_idx, seq_idx, dense_vals, table):
    sp2d = sparse_idx.reshape(B * NF // 128, 128)
    mesh = plsc.VectorSubcoreMesh(core_axis_name="c", subcore_axis_name="s")
    k = functools.partial(
        pl.kernel,
        mesh=mesh,
        compiler_params=pltpu.CompilerParams(use_tc_tiling_on_sc=False),
        out_type=[
            jax.ShapeDtypeStruct((B * NF, D), jnp.float32),
            jax.ShapeDtypeStruct((B, D), jnp.float32),
        ],
        scratch_types=[
            pltpu.VMEM((128,), jnp.int32),
            pltpu.VMEM((128, D), jnp.float32),
            pltpu.VMEM((BPW, L), jnp.int32),
            pltpu.VMEM((L, D), jnp.float32),
            pltpu.VMEM((BPW, D), jnp.float32),
            pltpu.SemaphoreType.DMA,
        ],
    )(_body)
    out_sp, out_pool = k(table, sp2d, seq_idx)
    return jnp.concatenate(
        [out_sp.reshape(B, NF * D), out_pool, dense_vals], axis=1)
